# baseline (speedup 1.0000x reference)
import functools
import jax
import jax.numpy as jnp
import numpy as np
from jax import lax
from jax.experimental import pallas as pl
from jax.experimental.pallas import tpu as pltpu
from jax.experimental.pallas import tpu_sc as plsc

N_NODES = 10000
N_GRAPHS = 64
D = 128
DH = 64
CHUNK = 128
NCORES = 2
NSUB = 16
ROW_BLK = 1000
AGG_ROWS = 10240


def _pack_bf16_halves(e):
    ub = lax.bitcast_convert_type(e, jnp.uint32)
    r16 = (ub + jnp.uint32(0x8000)) >> jnp.uint32(16)
    packs = []
    for g in range(D // 32):
        a = r16[:, 32 * g:32 * g + 16]
        b = r16[:, 32 * g + 16:32 * g + 32]
        packs.append(a | (b << jnp.uint32(16)))
    half0 = lax.bitcast_convert_type(jnp.concatenate(packs[:2], axis=1), jnp.int32)
    half1 = lax.bitcast_convert_type(jnp.concatenate(packs[2:], axis=1), jnp.int32)
    return half0, half1


def _edge_proj_body(ea_ref, We1_ref, be1_ref, We2_ref, be2_ref, e1_ref, e2_ref):
    ea = ea_ref[...]
    e1 = jnp.dot(ea, We1_ref[...], preferred_element_type=jnp.float32) + be1_ref[...]
    e2 = jnp.dot(ea, We2_ref[...], preferred_element_type=jnp.float32) + be2_ref[...]
    e1_ref[0], e1_ref[1] = _pack_bf16_halves(e1)
    e2_ref[0], e2_ref[1] = _pack_bf16_halves(e2)


def _edge_proj(ea_p, We1, be1, We2, be2, e_pad, blk):
    grid = e_pad // blk
    return pl.pallas_call(
        _edge_proj_body,
        grid=(grid,),
        in_specs=[
            pl.BlockSpec((blk, 3), lambda i: (i, 0)),
            pl.BlockSpec((3, D), lambda i: (0, 0)),
            pl.BlockSpec((1, D), lambda i: (0, 0)),
            pl.BlockSpec((3, D), lambda i: (0, 0)),
            pl.BlockSpec((1, D), lambda i: (0, 0)),
        ],
        out_specs=[
            pl.BlockSpec((2, blk, DH // 2), lambda i: (0, i, 0)),
            pl.BlockSpec((2, blk, DH // 2), lambda i: (0, i, 0)),
        ],
        out_shape=[
            jax.ShapeDtypeStruct((2, e_pad, DH // 2), jnp.int32),
            jax.ShapeDtypeStruct((2, e_pad, DH // 2), jnp.int32),
        ],
    )(ea_p, We1, be1, We2, be2)


def _edge_pass_body(n_chunks, x_hbm, e_hbm, src_hbm, dst_hbm, out_hbm,
                    idx_src, idx_dst, xg0, xg1, ev0, ev1, mv0, mv1, aggr_sh,
                    gs0, gs1, es0, es1, ss0, ss1):
    c = lax.axis_index("c")
    s = lax.axis_index("s")
    xg = (xg0, xg1)
    ev = (ev0, ev1)
    mv = (mv0, mv1)
    gsem = (gs0, gs1)
    esem = (es0, es1)
    ssem = (ss0, ss1)

    def zrow(r, _):
        for j in range(DH // 16):
            mv0[r, pl.ds(j * 16, 16)] = jnp.zeros((16,), jnp.float32)
        return 0
    lax.fori_loop(0, CHUNK, zrow, 0)
    for k in range(5):
        pltpu.sync_copy(mv0, aggr_sh.at[pl.ds(s * 640 + k * CHUNK, CHUNK)])
    plsc.subcore_barrier()

    pltpu.sync_copy(src_hbm.at[s], idx_src)
    pltpu.sync_copy(dst_hbm.at[s], idx_dst)
    coff = c * N_NODES

    def shift_row(g, _):
        for j in range(CHUNK // 16):
            sl = pl.ds(j * 16, 16)
            idx_src[g, sl] = idx_src[g, sl] + coff
        return 0
    lax.fori_loop(0, n_chunks, shift_row, 0)

    def issue_in(g, b):
        base = (s * n_chunks + g) * CHUNK
        pltpu.async_copy(x_hbm.at[idx_src.at[g]], xg[b], gsem[b])
        pltpu.async_copy(e_hbm.at[c, pl.ds(base, CHUNK)], ev[b], esem[b])

    def wait_in(g, b):
        base = (s * n_chunks + g) * CHUNK
        pltpu.make_async_copy(x_hbm.at[idx_src.at[g]], xg[b], gsem[b]).wait()
        pltpu.make_async_copy(e_hbm.at[c, pl.ds(base, CHUNK)], ev[b], esem[b]).wait()

    def wait_sc(g, b):
        pltpu.make_async_copy(mv[b], aggr_sh.at[idx_dst.at[g]], ssem[b]).wait()

    issue_in(0, 0)
    issue_in(1, 1)

    def pair_body(gp, _):
        for b in range(2):
            g = gp * 2 + b
            wait_in(g, b)

            @pl.when(g >= 2)
            def _():
                wait_sc(g - 2, b)

            @plsc.parallel_loop(0, CHUNK, step=1, unroll=8)
            def _(r):
                for grp in range(DH // 32):
                    ep = ev[b][r, pl.ds(16 * grp, 16)]
                    ea_ = lax.bitcast_convert_type(ep << 16, jnp.float32)
                    eb_ = lax.bitcast_convert_type(ep & jnp.int32(-65536), jnp.float32)
                    sa = pl.ds(32 * grp, 16)
                    sb = pl.ds(32 * grp + 16, 16)
                    mv[b][r, sa] = jnp.maximum(xg[b][r, sa] + ea_, 0.0)
                    mv[b][r, sb] = jnp.maximum(xg[b][r, sb] + eb_, 0.0)
            pltpu.async_copy(mv[b], aggr_sh.at[idx_dst.at[g]], ssem[b], add=True)

            @pl.when(g + 2 < n_chunks)
            def _():
                issue_in(g + 2, b)
        return 0
    lax.fori_loop(0, n_chunks // 2, pair_body, 0)
    wait_sc(n_chunks - 2, 0)
    wait_sc(n_chunks - 1, 1)

    plsc.subcore_barrier()
    for k in range(5):
        row = s * 640 + k * CHUNK
        pltpu.sync_copy(aggr_sh.at[pl.ds(row, CHUNK)], mv0)
        pltpu.sync_copy(mv0, out_hbm.at[c, pl.ds(row, CHUNK)])


def _edge_pass(x2, e2s, src_r, dst_r, n_chunks):
    mesh = plsc.VectorSubcoreMesh(core_axis_name="c", subcore_axis_name="s")
    kern = pl.kernel(
        functools.partial(_edge_pass_body, n_chunks),
        out_type=jax.ShapeDtypeStruct((NCORES, AGG_ROWS, DH), jnp.float32),
        mesh=mesh,
        compiler_params=pltpu.CompilerParams(use_tc_tiling_on_sc=False),
        scratch_types=[
            pltpu.VMEM((n_chunks, CHUNK), jnp.int32),
            pltpu.VMEM((n_chunks, CHUNK), jnp.int32),
            pltpu.VMEM((CHUNK, DH), jnp.float32),
            pltpu.VMEM((CHUNK, DH), jnp.float32),
            pltpu.VMEM((CHUNK, DH // 2), jnp.int32),
            pltpu.VMEM((CHUNK, DH // 2), jnp.int32),
            pltpu.VMEM((CHUNK, DH), jnp.float32),
            pltpu.VMEM((CHUNK, DH), jnp.float32),
            pltpu.VMEM_SHARED((AGG_ROWS, DH), jnp.float32),
            pltpu.SemaphoreType.DMA,
            pltpu.SemaphoreType.DMA,
            pltpu.SemaphoreType.DMA,
            pltpu.SemaphoreType.DMA,
            pltpu.SemaphoreType.DMA,
            pltpu.SemaphoreType.DMA,
        ],
    )
    return kern(x2, e2s, src_r, dst_r)


def _split_body(x_ref, o_ref):
    @pl.when(pl.program_id(0) == 0)
    def _():
        o_ref[...] = x_ref[...][:, :DH]

    @pl.when(pl.program_id(0) == 1)
    def _():
        o_ref[...] = x_ref[...][:, DH:]


def _split_rows(x):
    nb = N_NODES // ROW_BLK
    return pl.pallas_call(
        _split_body,
        grid=(2, nb),
        in_specs=[pl.BlockSpec((ROW_BLK, D), lambda h, i: (i, 0))],
        out_specs=pl.BlockSpec((ROW_BLK, DH), lambda h, i: (h * (N_NODES // ROW_BLK) + i, 0)),
        out_shape=jax.ShapeDtypeStruct((2 * N_NODES, DH), jnp.float32),
    )(x)


def _node_mlp_body(xl_ref, xh_ref, a_ref, Wa_ref, ba_ref, Wb_ref, bb_ref, h_ref):
    x = jnp.concatenate([xl_ref[...], xh_ref[...]], axis=-1)
    a = jnp.concatenate([a_ref[0], a_ref[1]], axis=-1)
    h0 = x + a
    t = jnp.maximum(jnp.dot(h0, Wa_ref[...], preferred_element_type=jnp.float32) + ba_ref[...], 0.0)
    h = jnp.maximum(jnp.dot(t, Wb_ref[...], preferred_element_type=jnp.float32) + bb_ref[...], 0.0)

    @pl.when(pl.program_id(0) == 0)
    def _():
        h_ref[...] = h[:, :DH]

    @pl.when(pl.program_id(0) == 1)
    def _():
        h_ref[...] = h[:, DH:]


def _node_mlp(x2f, aggr, Wa, ba, Wb, bb):
    nb = N_NODES // ROW_BLK
    return pl.pallas_call(
        _node_mlp_body,
        grid=(2, nb),
        in_specs=[
            pl.BlockSpec((ROW_BLK, DH), lambda h, i: (i, 0)),
            pl.BlockSpec((ROW_BLK, DH), lambda h, i: (N_NODES // ROW_BLK + i, 0)),
            pl.BlockSpec((2, ROW_BLK, DH), lambda h, i: (0, i, 0)),
            pl.BlockSpec((D, D), lambda h, i: (0, 0)),
            pl.BlockSpec((1, D), lambda h, i: (0, 0)),
            pl.BlockSpec((D, D), lambda h, i: (0, 0)),
            pl.BlockSpec((1, D), lambda h, i: (0, 0)),
        ],
        out_specs=pl.BlockSpec((ROW_BLK, DH), lambda h, i: (h * (N_NODES // ROW_BLK) + i, 0)),
        out_shape=jax.ShapeDtypeStruct((2 * N_NODES, DH), jnp.float32),
    )(x2f, x2f, aggr, Wa, ba, Wb, bb)


def _pool_mlp_body(xl_ref, xh_ref, a_ref, batch_ref, Wa_ref, ba_ref, Wb_ref, bb_ref,
                   Wm1_ref, bm1_ref, Wm2_ref, bm2_ref, o_ref, acc_ref):
    x = jnp.concatenate([xl_ref[...], xh_ref[...]], axis=-1)
    a = jnp.concatenate([a_ref[0], a_ref[1]], axis=-1)
    h0 = x + a
    t = jnp.maximum(jnp.dot(h0, Wa_ref[...], preferred_element_type=jnp.float32) + ba_ref[...], 0.0)
    h = jnp.maximum(jnp.dot(t, Wb_ref[...], preferred_element_type=jnp.float32) + bb_ref[...], 0.0)
    b = batch_ref[0, 0]
    oh = (lax.broadcasted_iota(jnp.int32, (N_GRAPHS, ROW_BLK), 0) == b[None, :]).astype(jnp.float32)
    part = jnp.dot(oh, h, preferred_element_type=jnp.float32)

    @pl.when(pl.program_id(0) == 0)
    def _():
        acc_ref[...] = jnp.zeros_like(acc_ref)
    acc_ref[...] += part

    @pl.when(pl.program_id(0) == pl.num_programs(0) - 1)
    def _():
        tm = jnp.maximum(jnp.dot(acc_ref[...], Wm1_ref[...], preferred_element_type=jnp.float32) + bm1_ref[...], 0.0)
        o_ref[...] = jnp.dot(tm, Wm2_ref[...], preferred_element_type=jnp.float32) + bm2_ref[...]


def _pool_mlp(h2, aggr, batch_r, Wa, ba, Wb, bb, Wm1, bm1, Wm2, bm2):
    grid = N_NODES // ROW_BLK
    wspec = pl.BlockSpec((D, D), lambda i: (0, 0))
    bspec = pl.BlockSpec((1, D), lambda i: (0, 0))
    return pl.pallas_call(
        _pool_mlp_body,
        grid=(grid,),
        in_specs=[
            pl.BlockSpec((ROW_BLK, DH), lambda i: (i, 0)),
            pl.BlockSpec((ROW_BLK, DH), lambda i: (N_NODES // ROW_BLK + i, 0)),
            pl.BlockSpec((2, ROW_BLK, DH), lambda i: (0, i, 0)),
            pl.BlockSpec((1, 1, ROW_BLK), lambda i: (i, 0, 0)),
            wspec, bspec, wspec, bspec,
            wspec, bspec, wspec, bspec,
        ],
        out_specs=pl.BlockSpec((N_GRAPHS, D), lambda i: (0, 0)),
        out_shape=jax.ShapeDtypeStruct((N_GRAPHS, D), jnp.float32),
        scratch_shapes=[pltpu.VMEM((N_GRAPHS, D), jnp.float32)],
    )(h2, h2, aggr, batch_r, Wa, ba, Wb, bb, Wm1, bm1, Wm2, bm2)


def kernel(x, edge_index, edge_attr, batch, We1, be1, W1a, b1a, W1b, b1b,
           We2, be2, W2a, b2a, W2b, b2b, Wm1, bm1, Wm2, bm2):
    E = edge_index.shape[1]
    n_chunks = (E + NSUB * CHUNK - 1) // (NSUB * CHUNK)
    n_chunks += n_chunks % 2
    e_pad = n_chunks * NSUB * CHUNK
    pad = e_pad - E

    src_r = jnp.concatenate([edge_index[0], jnp.zeros((pad,), jnp.int32)]).reshape(NSUB, n_chunks, CHUNK)
    dst_r = jnp.concatenate([edge_index[1], jnp.full((pad,), N_NODES, jnp.int32)]).reshape(NSUB, n_chunks, CHUNK)
    batch_r = batch.reshape(N_NODES // ROW_BLK, 1, ROW_BLK)

    x2f = _split_rows(x)
    e1, e2 = _edge_proj(edge_attr, We1, be1.reshape(1, D), We2, be2.reshape(1, D), e_pad, 4096)
    aggr1 = _edge_pass(x2f, e1, src_r, dst_r, n_chunks)
    h1 = _node_mlp(x2f, aggr1, W1a, b1a.reshape(1, D), W1b, b1b.reshape(1, D))

    aggr2 = _edge_pass(h1, e2, src_r, dst_r, n_chunks)
    pooled = _pool_mlp(h1, aggr2, batch_r, W2a, b2a.reshape(1, D), W2b, b2b.reshape(1, D),
                       Wm1, bm1.reshape(1, D), Wm2, bm2.reshape(1, D))
    return pooled


# speedup vs baseline: 1.0095x; 1.0095x over previous
import functools
import jax
import jax.numpy as jnp
import numpy as np
from jax import lax
from jax.experimental import pallas as pl
from jax.experimental.pallas import tpu as pltpu
from jax.experimental.pallas import tpu_sc as plsc

N_NODES = 10000
N_GRAPHS = 64
D = 128
DH = 64
CHUNK = 128
NCORES = 2
NSUB = 16
ROW_BLK = 1000
AGG_ROWS = 10240


def _pack_bf16_halves(e):
    ub = lax.bitcast_convert_type(e, jnp.uint32)
    r16 = (ub + jnp.uint32(0x8000)) >> jnp.uint32(16)
    packs = []
    for g in range(D // 32):
        a = r16[:, 32 * g:32 * g + 16]
        b = r16[:, 32 * g + 16:32 * g + 32]
        packs.append(a | (b << jnp.uint32(16)))
    half0 = lax.bitcast_convert_type(jnp.concatenate(packs[:2], axis=1), jnp.int32)
    half1 = lax.bitcast_convert_type(jnp.concatenate(packs[2:], axis=1), jnp.int32)
    return half0, half1


def _edge_proj_body(ea_ref, We1_ref, be1_ref, We2_ref, be2_ref, e1_ref, e2_ref):
    ea = ea_ref[...]
    e1 = jnp.dot(ea, We1_ref[...], preferred_element_type=jnp.float32) + be1_ref[...]
    e2 = jnp.dot(ea, We2_ref[...], preferred_element_type=jnp.float32) + be2_ref[...]
    e1_ref[0], e1_ref[1] = _pack_bf16_halves(e1)
    e2_ref[0], e2_ref[1] = _pack_bf16_halves(e2)


def _edge_proj(ea_p, We1, be1, We2, be2, e_pad, blk):
    grid = e_pad // blk
    return pl.pallas_call(
        _edge_proj_body,
        grid=(grid,),
        in_specs=[
            pl.BlockSpec((blk, 3), lambda i: (i, 0)),
            pl.BlockSpec((3, D), lambda i: (0, 0)),
            pl.BlockSpec((1, D), lambda i: (0, 0)),
            pl.BlockSpec((3, D), lambda i: (0, 0)),
            pl.BlockSpec((1, D), lambda i: (0, 0)),
        ],
        out_specs=[
            pl.BlockSpec((2, blk, DH // 2), lambda i: (0, i, 0)),
            pl.BlockSpec((2, blk, DH // 2), lambda i: (0, i, 0)),
        ],
        out_shape=[
            jax.ShapeDtypeStruct((2, e_pad, DH // 2), jnp.int32),
            jax.ShapeDtypeStruct((2, e_pad, DH // 2), jnp.int32),
        ],
    )(ea_p, We1, be1, We2, be2)


def _edge_pass_body(n_chunks, x_hbm, e_hbm, src_hbm, dst_hbm, out_hbm,
                    idx_src, idx_dst, xg0, xg1, ev0, ev1, mv0, mv1, aggr_sh,
                    gs0, gs1, es0, es1, ss0, ss1):
    c = lax.axis_index("c")
    s = lax.axis_index("s")
    xg = (xg0, xg1)
    ev = (ev0, ev1)
    mv = (mv0, mv1)
    gsem = (gs0, gs1)
    esem = (es0, es1)
    ssem = (ss0, ss1)

    def zrow(r, _):
        for j in range(DH // 16):
            mv0[r, pl.ds(j * 16, 16)] = jnp.zeros((16,), jnp.float32)
        return 0
    lax.fori_loop(0, CHUNK, zrow, 0)
    for k in range(5):
        pltpu.sync_copy(mv0, aggr_sh.at[pl.ds(s * 640 + k * CHUNK, CHUNK)])
    plsc.subcore_barrier()

    pltpu.sync_copy(src_hbm.at[s], idx_src)
    pltpu.sync_copy(dst_hbm.at[s], idx_dst)
    coff = c * N_NODES

    def shift_row(g, _):
        for j in range(CHUNK // 16):
            sl = pl.ds(j * 16, 16)
            idx_src[g, sl] = idx_src[g, sl] + coff
        return 0
    lax.fori_loop(0, n_chunks, shift_row, 0)

    def issue_in(g, b):
        base = (s * n_chunks + g) * CHUNK
        pltpu.async_copy(x_hbm.at[idx_src.at[g]], xg[b], gsem[b])
        pltpu.async_copy(e_hbm.at[c, pl.ds(base, CHUNK)], ev[b], esem[b])

    def wait_in(g, b):
        base = (s * n_chunks + g) * CHUNK
        pltpu.make_async_copy(x_hbm.at[idx_src.at[g]], xg[b], gsem[b]).wait()
        pltpu.make_async_copy(e_hbm.at[c, pl.ds(base, CHUNK)], ev[b], esem[b]).wait()

    def wait_sc(g, b):
        pltpu.make_async_copy(mv[b], aggr_sh.at[idx_dst.at[g]], ssem[b]).wait()

    issue_in(0, 0)
    issue_in(1, 1)

    def pair_body(gp, _):
        for b in range(2):
            g = gp * 2 + b
            wait_in(g, b)

            @pl.when(g >= 2)
            def _():
                wait_sc(g - 2, b)

            @plsc.parallel_loop(0, CHUNK, step=1, unroll=8)
            def _(r):
                for grp in range(DH // 32):
                    ep = ev[b][r, pl.ds(16 * grp, 16)]
                    ea_ = lax.bitcast_convert_type(ep << 16, jnp.float32)
                    eb_ = lax.bitcast_convert_type(ep & jnp.int32(-65536), jnp.float32)
                    sa = pl.ds(32 * grp, 16)
                    sb = pl.ds(32 * grp + 16, 16)
                    mv[b][r, sa] = jnp.maximum(xg[b][r, sa] + ea_, 0.0)
                    mv[b][r, sb] = jnp.maximum(xg[b][r, sb] + eb_, 0.0)
            pltpu.async_copy(mv[b], aggr_sh.at[idx_dst.at[g]], ssem[b], add=True)

            @pl.when(g + 2 < n_chunks)
            def _():
                issue_in(g + 2, b)
        return 0
    lax.fori_loop(0, n_chunks // 2, pair_body, 0)
    wait_sc(n_chunks - 2, 0)
    wait_sc(n_chunks - 1, 1)

    plsc.subcore_barrier()
    for k in range(5):
        row = s * 640 + k * CHUNK
        pltpu.sync_copy(aggr_sh.at[pl.ds(row, CHUNK)], mv0)
        pltpu.sync_copy(mv0, out_hbm.at[c, pl.ds(row, CHUNK)])


def _edge_pass(x2, e2s, src_r, dst_r, n_chunks):
    mesh = plsc.VectorSubcoreMesh(core_axis_name="c", subcore_axis_name="s")
    kern = pl.kernel(
        functools.partial(_edge_pass_body, n_chunks),
        out_type=jax.ShapeDtypeStruct((NCORES, AGG_ROWS, DH), jnp.float32),
        mesh=mesh,
        compiler_params=pltpu.CompilerParams(use_tc_tiling_on_sc=False),
        scratch_types=[
            pltpu.VMEM((n_chunks, CHUNK), jnp.int32),
            pltpu.VMEM((n_chunks, CHUNK), jnp.int32),
            pltpu.VMEM((CHUNK, DH), jnp.float32),
            pltpu.VMEM((CHUNK, DH), jnp.float32),
            pltpu.VMEM((CHUNK, DH // 2), jnp.int32),
            pltpu.VMEM((CHUNK, DH // 2), jnp.int32),
            pltpu.VMEM((CHUNK, DH), jnp.float32),
            pltpu.VMEM((CHUNK, DH), jnp.float32),
            pltpu.VMEM_SHARED((AGG_ROWS, DH), jnp.float32),
            pltpu.SemaphoreType.DMA,
            pltpu.SemaphoreType.DMA,
            pltpu.SemaphoreType.DMA,
            pltpu.SemaphoreType.DMA,
            pltpu.SemaphoreType.DMA,
            pltpu.SemaphoreType.DMA,
        ],
    )
    return kern(x2, e2s, src_r, dst_r)


def _split_body(x_ref, o_ref):
    @pl.when(pl.program_id(0) == 0)
    def _():
        o_ref[...] = x_ref[...][:, :DH]

    @pl.when(pl.program_id(0) == 1)
    def _():
        o_ref[...] = x_ref[...][:, DH:]


def _split_rows(x):
    nb = N_NODES // ROW_BLK
    return pl.pallas_call(
        _split_body,
        grid=(2, nb),
        in_specs=[pl.BlockSpec((ROW_BLK, D), lambda h, i: (i, 0))],
        out_specs=pl.BlockSpec((ROW_BLK, DH), lambda h, i: (h * (N_NODES // ROW_BLK) + i, 0)),
        out_shape=jax.ShapeDtypeStruct((2 * N_NODES, DH), jnp.float32),
    )(x)


def _node_mlp_body(xl_ref, xh_ref, a_ref, Wa_ref, ba_ref, Wb_ref, bb_ref, h_ref):
    x = jnp.concatenate([xl_ref[...], xh_ref[...]], axis=-1)
    a = jnp.concatenate([a_ref[0], a_ref[1]], axis=-1)
    h0 = x + a
    t = jnp.maximum(jnp.dot(h0, Wa_ref[...], preferred_element_type=jnp.float32) + ba_ref[...], 0.0)
    h = jnp.maximum(jnp.dot(t, Wb_ref[...], preferred_element_type=jnp.float32) + bb_ref[...], 0.0)
    h_ref[0] = h[:, :DH]
    h_ref[1] = h[:, DH:]


def _node_mlp(x2f, aggr, Wa, ba, Wb, bb):
    grid = N_NODES // ROW_BLK
    return pl.pallas_call(
        _node_mlp_body,
        grid=(grid,),
        in_specs=[
            pl.BlockSpec((ROW_BLK, DH), lambda i: (i, 0)),
            pl.BlockSpec((ROW_BLK, DH), lambda i: (N_NODES // ROW_BLK + i, 0)),
            pl.BlockSpec((2, ROW_BLK, DH), lambda i: (0, i, 0)),
            pl.BlockSpec((D, D), lambda i: (0, 0)),
            pl.BlockSpec((1, D), lambda i: (0, 0)),
            pl.BlockSpec((D, D), lambda i: (0, 0)),
            pl.BlockSpec((1, D), lambda i: (0, 0)),
        ],
        out_specs=pl.BlockSpec((2, ROW_BLK, DH), lambda i: (0, i, 0)),
        out_shape=jax.ShapeDtypeStruct((2, N_NODES, DH), jnp.float32),
    )(x2f, x2f, aggr, Wa, ba, Wb, bb)


def _pool_mlp_body(x_ref, a_ref, batch_ref, Wa_ref, ba_ref, Wb_ref, bb_ref,
                   Wm1_ref, bm1_ref, Wm2_ref, bm2_ref, o_ref, acc_ref):
    x = jnp.concatenate([x_ref[0], x_ref[1]], axis=-1)
    a = jnp.concatenate([a_ref[0], a_ref[1]], axis=-1)
    h0 = x + a
    t = jnp.maximum(jnp.dot(h0, Wa_ref[...], preferred_element_type=jnp.float32) + ba_ref[...], 0.0)
    h = jnp.maximum(jnp.dot(t, Wb_ref[...], preferred_element_type=jnp.float32) + bb_ref[...], 0.0)
    b = batch_ref[0, 0]
    oh = (lax.broadcasted_iota(jnp.int32, (N_GRAPHS, ROW_BLK), 0) == b[None, :]).astype(jnp.float32)
    part = jnp.dot(oh, h, preferred_element_type=jnp.float32)

    @pl.when(pl.program_id(0) == 0)
    def _():
        acc_ref[...] = jnp.zeros_like(acc_ref)
    acc_ref[...] += part

    @pl.when(pl.program_id(0) == pl.num_programs(0) - 1)
    def _():
        tm = jnp.maximum(jnp.dot(acc_ref[...], Wm1_ref[...], preferred_element_type=jnp.float32) + bm1_ref[...], 0.0)
        o_ref[...] = jnp.dot(tm, Wm2_ref[...], preferred_element_type=jnp.float32) + bm2_ref[...]


def _pool_mlp(h2, aggr, batch_r, Wa, ba, Wb, bb, Wm1, bm1, Wm2, bm2):
    grid = N_NODES // ROW_BLK
    wspec = pl.BlockSpec((D, D), lambda i: (0, 0))
    bspec = pl.BlockSpec((1, D), lambda i: (0, 0))
    return pl.pallas_call(
        _pool_mlp_body,
        grid=(grid,),
        in_specs=[
            pl.BlockSpec((2, ROW_BLK, DH), lambda i: (0, i, 0)),
            pl.BlockSpec((2, ROW_BLK, DH), lambda i: (0, i, 0)),
            pl.BlockSpec((1, 1, ROW_BLK), lambda i: (i, 0, 0)),
            wspec, bspec, wspec, bspec,
            wspec, bspec, wspec, bspec,
        ],
        out_specs=pl.BlockSpec((N_GRAPHS, D), lambda i: (0, 0)),
        out_shape=jax.ShapeDtypeStruct((N_GRAPHS, D), jnp.float32),
        scratch_shapes=[pltpu.VMEM((N_GRAPHS, D), jnp.float32)],
    )(h2, aggr, batch_r, Wa, ba, Wb, bb, Wm1, bm1, Wm2, bm2)


def kernel(x, edge_index, edge_attr, batch, We1, be1, W1a, b1a, W1b, b1b,
           We2, be2, W2a, b2a, W2b, b2b, Wm1, bm1, Wm2, bm2):
    E = edge_index.shape[1]
    n_chunks = (E + NSUB * CHUNK - 1) // (NSUB * CHUNK)
    n_chunks += n_chunks % 2
    e_pad = n_chunks * NSUB * CHUNK
    pad = e_pad - E

    src_r = jnp.concatenate([edge_index[0], jnp.zeros((pad,), jnp.int32)]).reshape(NSUB, n_chunks, CHUNK)
    dst_r = jnp.concatenate([edge_index[1], jnp.full((pad,), N_NODES, jnp.int32)]).reshape(NSUB, n_chunks, CHUNK)
    batch_r = batch.reshape(N_NODES // ROW_BLK, 1, ROW_BLK)

    x2f = _split_rows(x)
    e1, e2 = _edge_proj(edge_attr, We1, be1.reshape(1, D), We2, be2.reshape(1, D), e_pad, 4096)
    aggr1 = _edge_pass(x2f, e1, src_r, dst_r, n_chunks)
    h1 = _node_mlp(x2f, aggr1, W1a, b1a.reshape(1, D), W1b, b1b.reshape(1, D))

    aggr2 = _edge_pass(h1.reshape(2 * N_NODES, DH), e2, src_r, dst_r, n_chunks)
    pooled = _pool_mlp(h1, aggr2, batch_r, W2a, b2a.reshape(1, D), W2b, b2b.reshape(1, D),
                       Wm1, bm1.reshape(1, D), Wm2, bm2.reshape(1, D))
    return pooled
